# baseline (speedup 1.0000x reference)
import functools

import jax
import jax.numpy as jnp
from jax import lax
from jax.experimental import pallas as pl
from jax.experimental.pallas import tpu as pltpu
from jax.experimental.pallas import tpu_sc as plsc

BATCH = 16384
TEXT_DIM = 128
FINAL_DIM = 64
HIDDEN = (TEXT_DIM + 1) // 2
NUM_USERS = 1000000
HALF_USERS = NUM_USERS // 2

NUM_CORES = 2
NUM_SUBCORES = 16
NW = NUM_CORES * NUM_SUBCORES
BPW = BATCH // NW
CHUNK = 128
NCH = BPW // CHUNK


def _tc_item_gather_body(ids_ref, tab_ref, out_ref, sem):
  nrows = out_ref.shape[0]

  def row(i, carry):
    r = ids_ref[0, 0, i]
    pltpu.make_async_copy(
        tab_ref.at[pl.ds(r, 1)], out_ref.at[pl.ds(i, 1)], sem).start()
    return carry

  lax.fori_loop(0, nrows, row, 0, unroll=8)
  pltpu.make_async_copy(tab_ref.at[pl.ds(0, nrows)], out_ref, sem).wait()


def _tc_item_gather(ids2d, table, block_m=2048):
  grid = (BATCH // block_m,)
  return pl.pallas_call(
      _tc_item_gather_body,
      grid=grid,
      in_specs=[
          pl.BlockSpec((1, 1, block_m), lambda i: (i, 0, 0),
                       memory_space=pltpu.SMEM),
          pl.BlockSpec(memory_space=pl.ANY),
      ],
      out_specs=pl.BlockSpec((block_m, TEXT_DIM), lambda i: (i, 0)),
      out_shape=jax.ShapeDtypeStruct((BATCH, TEXT_DIM), jnp.float32),
      scratch_shapes=[pltpu.SemaphoreType.DMA],
  )(ids2d, table)


def _sc_user_gather(uids2d, utab3):
  mesh = plsc.VectorSubcoreMesh(core_axis_name="c", subcore_axis_name="s")

  @functools.partial(
      pl.kernel,
      out_type=jax.ShapeDtypeStruct((BATCH, FINAL_DIM), jnp.float32),
      mesh=mesh,
      scratch_types=[
          pltpu.VMEM((1, BPW), jnp.int32),
          pltpu.VMEM((BPW, FINAL_DIM), jnp.float32),
          pltpu.SemaphoreType.DMA,
      ],
  )
  def k(uids_hbm, utab_hbm, uout_hbm, uidx_v, urows, usem):
    wid = lax.axis_index("s") * NUM_CORES + lax.axis_index("c")
    pltpu.sync_copy(uids_hbm.at[pl.ds(wid, 1)], uidx_v)

    def row_dma_group(g, carry):
      v = uidx_v[0, pl.ds(g * 16, 16)]
      base16 = g * 16
      for lane in range(16):
        r = v[lane]
        h = r // HALF_USERS
        rr = r % HALF_USERS
        pltpu.async_copy(
            utab_hbm.at[h].at[pl.ds(rr, 1)],
            urows.at[pl.ds(base16 + lane, 1)], usem)
      return carry

    lax.fori_loop(0, BPW // 16, row_dma_group, 0)
    pltpu.make_async_copy(
        utab_hbm.at[0].at[pl.ds(0, BPW)], urows, usem).wait()
    pltpu.sync_copy(urows, uout_hbm.at[pl.ds(wid * BPW, BPW)])

  return k(uids2d, utab3)


def _mlp_body(x_ref, p_ref, w1m_ref, w1l_ref, b1_ref, w2t_ref, b2_ref, o_ref):
  h = jnp.dot(x_ref[...], w1m_ref[...], preferred_element_type=jnp.float32)
  h = h + p_ref[...] * w1l_ref[...] + b1_ref[...]
  h = jnp.maximum(h, 0.0)
  out = (jnp.dot(h, w2t_ref[...], preferred_element_type=jnp.float32)
         + b2_ref[...])
  o_ref[...] = out.T


def _mlp(text_vecs, prices_col, w1m, w1l, b1r, w2t, b2r, block_m=2048):
  grid = (BATCH // block_m,)
  return pl.pallas_call(
      _mlp_body,
      grid=grid,
      in_specs=[
          pl.BlockSpec((block_m, TEXT_DIM), lambda i: (i, 0)),
          pl.BlockSpec((block_m, 1), lambda i: (i, 0)),
          pl.BlockSpec((TEXT_DIM, HIDDEN), lambda i: (0, 0)),
          pl.BlockSpec((1, HIDDEN), lambda i: (0, 0)),
          pl.BlockSpec((1, HIDDEN), lambda i: (0, 0)),
          pl.BlockSpec((HIDDEN, FINAL_DIM), lambda i: (0, 0)),
          pl.BlockSpec((1, FINAL_DIM), lambda i: (0, 0)),
      ],
      out_specs=pl.BlockSpec((FINAL_DIM, block_m), lambda i: (0, i)),
      out_shape=jax.ShapeDtypeStruct((FINAL_DIM, BATCH), jnp.float32),
  )(text_vecs, prices_col, w1m, w1l, b1r, w2t, b2r)


def kernel(user_ids, item_ids, item_prices, user_table, item_text_table,
           W1, b1, W2, b2):
  uids2 = user_ids.astype(jnp.int32).reshape(NW, BPW)
  iids2 = item_ids.astype(jnp.int32).reshape(BATCH // 2048, 1, 2048)
  text_vecs = _tc_item_gather(iids2, item_text_table)
  utab3 = user_table.reshape(2, HALF_USERS, FINAL_DIM)
  user_vec = _sc_user_gather(uids2, utab3)
  w1m = W1[:, :TEXT_DIM].T
  w1l = W1[:, TEXT_DIM:].T
  item_vec_t = _mlp(text_vecs, item_prices.reshape(BATCH, 1), w1m, w1l,
                    b1.reshape(1, HIDDEN), W2.T, b2.reshape(1, FINAL_DIM))
  return user_vec, item_vec_t.T


# speedup vs baseline: 1.2105x; 1.2105x over previous
import functools

import jax
import jax.numpy as jnp
from jax import lax
from jax.experimental import pallas as pl
from jax.experimental.pallas import tpu as pltpu
from jax.experimental.pallas import tpu_sc as plsc

BATCH = 16384
TEXT_DIM = 128
FINAL_DIM = 64
HIDDEN = (TEXT_DIM + 1) // 2
NUM_USERS = 1000000
HALF_USERS = NUM_USERS // 2

NUM_CORES = 2
NUM_SUBCORES = 16
NW = NUM_CORES * NUM_SUBCORES
BPW = BATCH // NW
CHUNK = 128
NCH = BPW // CHUNK


def _sc_gathers(uids2d, iids2d, utab3, item_text_table):
  mesh = plsc.VectorSubcoreMesh(core_axis_name="c", subcore_axis_name="s")

  @functools.partial(
      pl.kernel,
      out_type=(
          jax.ShapeDtypeStruct((BATCH, FINAL_DIM), jnp.float32),
          jax.ShapeDtypeStruct((BATCH, TEXT_DIM), jnp.float32),
      ),
      mesh=mesh,
      scratch_types=[
          pltpu.VMEM((NCH, CHUNK), jnp.int32),
          pltpu.VMEM((1, BPW), jnp.int32),
          pltpu.VMEM((BPW, FINAL_DIM), jnp.float32),
          pltpu.VMEM((CHUNK, TEXT_DIM), jnp.float32),
          pltpu.SemaphoreType.DMA,
          pltpu.SemaphoreType.DMA,
      ],
  )
  def k(uids_hbm, iids_hbm, utab_hbm, itab_hbm, uout_hbm, tout_hbm,
        iidx, uidx_v, urows, irows, isem, usem):
    wid = lax.axis_index("s") * NUM_CORES + lax.axis_index("c")
    row0 = wid * NCH
    pltpu.sync_copy(iids_hbm.at[pl.ds(row0, NCH)], iidx)
    pltpu.sync_copy(uids_hbm.at[pl.ds(wid, 1)], uidx_v)

    def row_dma_group(g, carry):
      v = uidx_v[0, pl.ds(g * 16, 16)]
      base16 = g * 16
      for lane in range(16):
        r = v[lane]
        h = r // HALF_USERS
        rr = r % HALF_USERS
        pltpu.async_copy(
            utab_hbm.at[h].at[pl.ds(rr, 1)],
            urows.at[pl.ds(base16 + lane, 1)], usem)
      return carry

    lax.fori_loop(0, BPW // 16, row_dma_group, 0)
    base = wid * BPW
    for j in range(NCH):
      pltpu.async_copy(itab_hbm.at[iidx.at[j]], irows, isem).wait()
      pltpu.sync_copy(irows, tout_hbm.at[pl.ds(base + j * CHUNK, CHUNK)])
    pltpu.make_async_copy(
        utab_hbm.at[0].at[pl.ds(0, BPW)], urows, usem).wait()
    pltpu.sync_copy(urows, uout_hbm.at[pl.ds(base, BPW)])

  return k(uids2d, iids2d, utab3, item_text_table)


def _mlp_body(x_ref, p_ref, w1m_ref, w1l_ref, b1_ref, w2t_ref, b2_ref, o_ref):
  h = jnp.dot(x_ref[...], w1m_ref[...], preferred_element_type=jnp.float32)
  h = h + p_ref[...] * w1l_ref[...] + b1_ref[...]
  h = jnp.maximum(h, 0.0)
  out = (jnp.dot(h, w2t_ref[...], preferred_element_type=jnp.float32)
         + b2_ref[...])
  o_ref[...] = out.T


def _mlp(text_vecs, prices_col, w1m, w1l, b1r, w2t, b2r, block_m=2048):
  grid = (BATCH // block_m,)
  return pl.pallas_call(
      _mlp_body,
      grid=grid,
      in_specs=[
          pl.BlockSpec((block_m, TEXT_DIM), lambda i: (i, 0)),
          pl.BlockSpec((block_m, 1), lambda i: (i, 0)),
          pl.BlockSpec((TEXT_DIM, HIDDEN), lambda i: (0, 0)),
          pl.BlockSpec((1, HIDDEN), lambda i: (0, 0)),
          pl.BlockSpec((1, HIDDEN), lambda i: (0, 0)),
          pl.BlockSpec((HIDDEN, FINAL_DIM), lambda i: (0, 0)),
          pl.BlockSpec((1, FINAL_DIM), lambda i: (0, 0)),
      ],
      out_specs=pl.BlockSpec((FINAL_DIM, block_m), lambda i: (0, i)),
      out_shape=jax.ShapeDtypeStruct((FINAL_DIM, BATCH), jnp.float32),
  )(text_vecs, prices_col, w1m, w1l, b1r, w2t, b2r)


def kernel(user_ids, item_ids, item_prices, user_table, item_text_table,
           W1, b1, W2, b2):
  uids2 = user_ids.astype(jnp.int32).reshape(NW, BPW)
  iids2 = item_ids.astype(jnp.int32).reshape(BATCH // CHUNK, CHUNK)
  utab3 = user_table.reshape(2, HALF_USERS, FINAL_DIM)
  user_vec, text_vecs = _sc_gathers(uids2, iids2, utab3, item_text_table)
  w1m = W1[:, :TEXT_DIM].T
  w1l = W1[:, TEXT_DIM:].T
  item_vec_t = _mlp(text_vecs, item_prices.reshape(BATCH, 1), w1m, w1l,
                    b1.reshape(1, HIDDEN), W2.T, b2.reshape(1, FINAL_DIM))
  return user_vec, item_vec_t.T


# speedup vs baseline: 1.2489x; 1.0317x over previous
import functools

import jax
import jax.numpy as jnp
from jax import lax
from jax.experimental import pallas as pl
from jax.experimental.pallas import tpu as pltpu
from jax.experimental.pallas import tpu_sc as plsc

BATCH = 16384
TEXT_DIM = 128
FINAL_DIM = 64
HIDDEN = (TEXT_DIM + 1) // 2
NUM_USERS = 1000000
HALF_USERS = NUM_USERS // 2

NUM_CORES = 2
NUM_SUBCORES = 16
NW = NUM_CORES * NUM_SUBCORES
BPW = BATCH // NW
CHUNK = 128
NCH = BPW // CHUNK


def _sc_item_gather(ids2d, table):
  mesh = plsc.VectorSubcoreMesh(core_axis_name="c", subcore_axis_name="s")

  @functools.partial(
      pl.kernel,
      out_type=jax.ShapeDtypeStruct((BATCH, TEXT_DIM), jnp.float32),
      mesh=mesh,
      scratch_types=[
          pltpu.VMEM((NCH, CHUNK), jnp.int32),
          pltpu.VMEM((BPW, TEXT_DIM), jnp.float32),
          pltpu.SemaphoreType.DMA,
      ],
  )
  def k(ids_hbm, tab_hbm, out_hbm, idx, rows, sem):
    wid = lax.axis_index("s") * NUM_CORES + lax.axis_index("c")
    row0 = wid * NCH
    pltpu.sync_copy(ids_hbm.at[pl.ds(row0, NCH)], idx)
    heads = []
    for j in range(NCH):
      heads.append(pltpu.async_copy(
          tab_hbm.at[idx.at[j]], rows.at[pl.ds(j * CHUNK, CHUNK)], sem))
    for h in heads:
      h.wait()
    pltpu.sync_copy(rows, out_hbm.at[pl.ds(wid * BPW, BPW)])

  return k(ids2d, table)


def _sc_user_gather(uids2d, utab3):
  mesh = plsc.VectorSubcoreMesh(core_axis_name="c", subcore_axis_name="s")

  @functools.partial(
      pl.kernel,
      out_type=jax.ShapeDtypeStruct((BATCH, FINAL_DIM), jnp.float32),
      mesh=mesh,
      scratch_types=[
          pltpu.VMEM((1, BPW), jnp.int32),
          pltpu.VMEM((BPW, FINAL_DIM), jnp.float32),
          pltpu.SemaphoreType.DMA,
      ],
  )
  def k(uids_hbm, utab_hbm, uout_hbm, uidx_v, urows, usem):
    wid = lax.axis_index("s") * NUM_CORES + lax.axis_index("c")
    pltpu.sync_copy(uids_hbm.at[pl.ds(wid, 1)], uidx_v)

    def row_dma_group(g, carry):
      v = uidx_v[0, pl.ds(g * 16, 16)]
      base16 = g * 16
      for lane in range(16):
        r = v[lane]
        h = r // HALF_USERS
        rr = r % HALF_USERS
        pltpu.async_copy(
            utab_hbm.at[h].at[pl.ds(rr, 1)],
            urows.at[pl.ds(base16 + lane, 1)], usem)
      return carry

    lax.fori_loop(0, BPW // 16, row_dma_group, 0)
    pltpu.make_async_copy(
        utab_hbm.at[0].at[pl.ds(0, BPW)], urows, usem).wait()
    pltpu.sync_copy(urows, uout_hbm.at[pl.ds(wid * BPW, BPW)])

  return k(uids2d, utab3)


def _mlp_body(x_ref, p_ref, w1m_ref, w1l_ref, b1_ref, w2t_ref, b2_ref, o_ref):
  h = jnp.dot(x_ref[...], w1m_ref[...], preferred_element_type=jnp.float32)
  h = h + p_ref[...] * w1l_ref[...] + b1_ref[...]
  h = jnp.maximum(h, 0.0)
  out = (jnp.dot(h, w2t_ref[...], preferred_element_type=jnp.float32)
         + b2_ref[...])
  o_ref[...] = out.T


def _mlp(text_vecs, prices_col, w1m, w1l, b1r, w2t, b2r, block_m=2048):
  grid = (BATCH // block_m,)
  return pl.pallas_call(
      _mlp_body,
      grid=grid,
      in_specs=[
          pl.BlockSpec((block_m, TEXT_DIM), lambda i: (i, 0)),
          pl.BlockSpec((block_m, 1), lambda i: (i, 0)),
          pl.BlockSpec((TEXT_DIM, HIDDEN), lambda i: (0, 0)),
          pl.BlockSpec((1, HIDDEN), lambda i: (0, 0)),
          pl.BlockSpec((1, HIDDEN), lambda i: (0, 0)),
          pl.BlockSpec((HIDDEN, FINAL_DIM), lambda i: (0, 0)),
          pl.BlockSpec((1, FINAL_DIM), lambda i: (0, 0)),
      ],
      out_specs=pl.BlockSpec((FINAL_DIM, block_m), lambda i: (0, i)),
      out_shape=jax.ShapeDtypeStruct((FINAL_DIM, BATCH), jnp.float32),
  )(text_vecs, prices_col, w1m, w1l, b1r, w2t, b2r)


def kernel(user_ids, item_ids, item_prices, user_table, item_text_table,
           W1, b1, W2, b2):
  uids2 = user_ids.astype(jnp.int32).reshape(NW, BPW)
  iids2 = item_ids.astype(jnp.int32).reshape(BATCH // CHUNK, CHUNK)
  text_vecs = _sc_item_gather(iids2, item_text_table)
  utab3 = user_table.reshape(2, HALF_USERS, FINAL_DIM)
  user_vec = _sc_user_gather(uids2, utab3)
  w1m = W1[:, :TEXT_DIM].T
  w1l = W1[:, TEXT_DIM:].T
  item_vec_t = _mlp(text_vecs, item_prices.reshape(BATCH, 1), w1m, w1l,
                    b1.reshape(1, HIDDEN), W2.T, b2.reshape(1, FINAL_DIM))
  return user_vec, item_vec_t.T


# speedup vs baseline: 1.4059x; 1.1257x over previous
import functools

import jax
import jax.numpy as jnp
from jax import lax
from jax.experimental import pallas as pl
from jax.experimental.pallas import tpu as pltpu
from jax.experimental.pallas import tpu_sc as plsc

BATCH = 16384
TEXT_DIM = 128
FINAL_DIM = 64
HIDDEN = (TEXT_DIM + 1) // 2
NUM_USERS = 1000000
HALF_USERS = NUM_USERS // 2

NUM_CORES = 2
NUM_SUBCORES = 16
NW = NUM_CORES * NUM_SUBCORES
BPW = BATCH // NW
CHUNK = 128
NCH = BPW // CHUNK


def _sc_item_gather(ids2d, table):
  mesh = plsc.VectorSubcoreMesh(core_axis_name="c", subcore_axis_name="s")

  @functools.partial(
      pl.kernel,
      out_type=jax.ShapeDtypeStruct((BATCH, TEXT_DIM), jnp.float32),
      mesh=mesh,
      scratch_types=[
          pltpu.VMEM((NCH, CHUNK), jnp.int32),
          pltpu.VMEM((BPW, TEXT_DIM), jnp.float32),
          pltpu.SemaphoreType.DMA,
      ],
  )
  def k(ids_hbm, tab_hbm, out_hbm, idx, rows, sem):
    wid = lax.axis_index("s") * NUM_CORES + lax.axis_index("c")
    row0 = wid * NCH
    pltpu.sync_copy(ids_hbm.at[pl.ds(row0, NCH)], idx)
    heads = []
    for j in range(NCH):
      heads.append(pltpu.async_copy(
          tab_hbm.at[idx.at[j]], rows.at[pl.ds(j * CHUNK, CHUNK)], sem))
    for h in heads:
      h.wait()
    pltpu.sync_copy(rows, out_hbm.at[pl.ds(wid * BPW, BPW)])

  return k(ids2d, table)


RING = 4
HPW = BPW // 2


def _sc_user_gather(uids2d, utab_t):
  mesh = plsc.VectorSubcoreMesh(core_axis_name="c", subcore_axis_name="s")

  @functools.partial(
      pl.kernel,
      out_type=jax.ShapeDtypeStruct((BATCH, FINAL_DIM), jnp.float32),
      mesh=mesh,
      compiler_params=pltpu.CompilerParams(needs_layout_passes=False),
      scratch_types=[
          pltpu.VMEM((1, BPW), jnp.int32),
          pltpu.VMEM((RING, FINAL_DIM, 128), jnp.float32),
          pltpu.VMEM((HPW, FINAL_DIM), jnp.float32),
          pltpu.SemaphoreType.DMA,
      ],
  )
  def k(uids_hbm, utab_hbm, uout_hbm, uidx_v, win, urows, usem):
    wid = lax.axis_index("s") * NUM_CORES + lax.axis_index("c")
    pltpu.sync_copy(uids_hbm.at[pl.ds(wid, 1)], uidx_v)
    iota = lax.iota(jnp.int32, 16)
    fvecs = [k16 * 16 + iota for k16 in range(4)]

    def fire(r, slot):
      w0 = pl.multiple_of((r >> 7) * 128, 128)
      pltpu.async_copy(utab_hbm.at[:, pl.ds(w0, 128)], win.at[slot], usem)

    v0 = uidx_v[0, pl.ds(0, 16)]
    for s in range(RING):
      fire(v0[s], s)

    ngroups = HPW // 16

    for p in range(2):
      def group(g, carry):
        v = uidx_v[0, pl.ds(p * HPW + g * 16, 16)]
        for q in range(4):
          for s in range(RING):
            pltpu.make_async_copy(
                utab_hbm.at[:, pl.ds(0, 128)], win.at[0], usem).wait()
            r = v[q * RING + s]
            c = jnp.broadcast_to(r & 127, (16,))
            i = g * 16 + q * RING + s
            for k16 in range(4):
              vals = plsc.load_gather(win.at[s], [fvecs[k16], c])
              urows[i, pl.ds(k16 * 16, 16)] = vals
            if q < 3:
              fire(v[(q + 1) * RING + s], s)
            elif p == 0:
              vn = uidx_v[0, pl.ds(p * HPW + (g + 1) * 16, 16)]
              fire(vn[s], s)
            else:
              @pl.when(g < ngroups - 1)
              def _():
                vn = uidx_v[0, pl.ds(p * HPW + (g + 1) * 16, 16)]
                fire(vn[s], s)

        return carry

      lax.fori_loop(0, ngroups, group, 0)
      pltpu.sync_copy(urows, uout_hbm.at[pl.ds(wid * BPW + p * HPW, HPW)])

  return k(uids2d, utab_t)


def _mlp_body(x_ref, p_ref, w1m_ref, w1l_ref, b1_ref, w2t_ref, b2_ref, o_ref):
  h = jnp.dot(x_ref[...], w1m_ref[...], preferred_element_type=jnp.float32)
  h = h + p_ref[...] * w1l_ref[...] + b1_ref[...]
  h = jnp.maximum(h, 0.0)
  out = (jnp.dot(h, w2t_ref[...], preferred_element_type=jnp.float32)
         + b2_ref[...])
  o_ref[...] = out.T


def _mlp(text_vecs, prices_col, w1m, w1l, b1r, w2t, b2r, block_m=2048):
  grid = (BATCH // block_m,)
  return pl.pallas_call(
      _mlp_body,
      grid=grid,
      in_specs=[
          pl.BlockSpec((block_m, TEXT_DIM), lambda i: (i, 0)),
          pl.BlockSpec((block_m, 1), lambda i: (i, 0)),
          pl.BlockSpec((TEXT_DIM, HIDDEN), lambda i: (0, 0)),
          pl.BlockSpec((1, HIDDEN), lambda i: (0, 0)),
          pl.BlockSpec((1, HIDDEN), lambda i: (0, 0)),
          pl.BlockSpec((HIDDEN, FINAL_DIM), lambda i: (0, 0)),
          pl.BlockSpec((1, FINAL_DIM), lambda i: (0, 0)),
      ],
      out_specs=pl.BlockSpec((FINAL_DIM, block_m), lambda i: (0, i)),
      out_shape=jax.ShapeDtypeStruct((FINAL_DIM, BATCH), jnp.float32),
  )(text_vecs, prices_col, w1m, w1l, b1r, w2t, b2r)


def kernel(user_ids, item_ids, item_prices, user_table, item_text_table,
           W1, b1, W2, b2):
  uids2 = user_ids.astype(jnp.int32).reshape(NW, BPW)
  iids2 = item_ids.astype(jnp.int32).reshape(BATCH // CHUNK, CHUNK)
  text_vecs = _sc_item_gather(iids2, item_text_table)
  user_vec = _sc_user_gather(uids2, user_table.T)
  w1m = W1[:, :TEXT_DIM].T
  w1l = W1[:, TEXT_DIM:].T
  item_vec_t = _mlp(text_vecs, item_prices.reshape(BATCH, 1), w1m, w1l,
                    b1.reshape(1, HIDDEN), W2.T, b2.reshape(1, FINAL_DIM))
  return user_vec, item_vec_t.T
